# baseline (speedup 1.0000x reference)
import functools

import jax
import jax.numpy as jnp
import numpy as np
from jax import lax
from jax.experimental import pallas as pl
from jax.experimental.pallas import tpu as pltpu
from jax.experimental.pallas import tpu_sc as plsc

_EPS = 1e-5
_BNS = float(1.0 / np.sqrt(1.0 + _EPS))
_NC = 2
_NS = 16
_CH = 128
_F32 = jnp.float32

_mesh = plsc.VectorSubcoreMesh(
    core_axis_name="c", subcore_axis_name="s", num_cores=_NC, num_subcores=_NS)


def _sc_gather(x, rid2, cid2, *, epad):
    nw = _NC * _NS
    per_w = epad // nw
    nch = per_w // _CH

    gch = 2 * _CH
    ncg = per_w // gch
    assert per_w % gch == 0 and ncg >= 6

    def body(x_hbm, rid_hbm, cid_hbm, xr_hbm, xc_hbm,
             ridv, cidv, b0, b1, b2, s0, s1, s2):
        w = lax.axis_index("s") * _NC + lax.axis_index("c")
        base = w * per_w
        pltpu.sync_copy(rid_hbm.at[pl.ds(base, per_w)], ridv)
        pltpu.sync_copy(cid_hbm.at[pl.ds(base, per_w)], cidv)
        bufs = (b0, b1, b2)
        sems = (s0, s1, s2)

        def phase(idxref, out_hbm):
            def issue(t, b):
                pltpu.async_copy(x_hbm.at[idxref.at[pl.ds(gch * t, gch)]],
                                 bufs[b], sems[b])

            def drain(t, b):
                pltpu.make_async_copy(x_hbm.at[idxref.at[pl.ds(gch * t, gch)]],
                                      bufs[b], sems[b]).wait()
                pltpu.sync_copy(bufs[b], out_hbm.at[pl.ds(base + gch * t, gch)])

            for b in range(3):
                issue(b, b)
            main = (ncg - 3) // 3

            def ring(tt, carry):
                for b in range(3):
                    t = 3 * tt + b
                    drain(t, b)
                    issue(t + 3, b)
                return carry

            lax.fori_loop(0, main, ring, 0)
            nxt = 3 * main + 3
            for t in range(3 * main, ncg):
                b = t % 3
                drain(t, b)
                if nxt < ncg:
                    issue(nxt, b)
                    nxt += 1

        phase(ridv, xr_hbm)
        phase(cidv, xc_hbm)

    call = pl.kernel(
        body,
        out_type=[jax.ShapeDtypeStruct((epad, 128), _F32),
                  jax.ShapeDtypeStruct((epad, 128), _F32)],
        mesh=_mesh,
        name="sc_gather",
        scratch_types=[
            pltpu.VMEM((per_w,), jnp.int32),
            pltpu.VMEM((per_w,), jnp.int32),
            pltpu.VMEM((gch, 128), _F32),
            pltpu.VMEM((gch, 128), _F32),
            pltpu.VMEM((gch, 128), _F32),
            pltpu.SemaphoreType.DMA,
            pltpu.SemaphoreType.DMA,
            pltpu.SemaphoreType.DMA,
        ],
    )
    return call(x, rid2, cid2)


def _sc_scatter(g0, g1, cid2, zacc, *, n, epad):
    per_s = epad // _NS
    nch = per_s // _CH
    nsp = zacc.shape[0]
    rows_w = nsp // _NS
    full = n // 640
    rem = n - full * 640

    def body(g0_hbm, g1_hbm, cid_hbm, z_hbm, s0_hbm, s1_hbm,
             acc, idxv, bufa, bufb, sema, semb):
        c = lax.axis_index("c")
        s = lax.axis_index("s")
        pltpu.sync_copy(z_hbm.at[pl.ds(s * rows_w, rows_w)],
                        acc.at[pl.ds(s * rows_w, rows_w)])
        pltpu.sync_copy(cid_hbm.at[pl.ds(s * nch, nch)], idxv)
        plsc.subcore_barrier()

        def run(g_hbm, out_hbm):
            base = s * per_s

            def pair(jj, carry):
                r0 = base + (2 * jj) * _CH
                ca = pltpu.async_copy(g_hbm.at[pl.ds(r0, _CH)], bufa, sema)
                cb = pltpu.async_copy(g_hbm.at[pl.ds(r0 + _CH, _CH)], bufb, semb)
                ca.wait()
                pltpu.sync_copy(bufa, acc.at[idxv.at[2 * jj]], add=True)
                cb.wait()
                pltpu.sync_copy(bufb, acc.at[idxv.at[2 * jj + 1]], add=True)
                return carry

            lax.fori_loop(0, nch // 2, pair, 0)
            plsc.subcore_barrier()

            @pl.when(s < full)
            def _():
                pltpu.sync_copy(acc.at[pl.ds(s * 640, 640)],
                                out_hbm.at[pl.ds(s * 640, 640)])

            if rem:
                @pl.when(s == full)
                def _():
                    pltpu.sync_copy(acc.at[pl.ds(full * 640, rem)],
                                    out_hbm.at[pl.ds(full * 640, rem)])

        @pl.when(c == 0)
        def _():
            run(g0_hbm, s0_hbm)

        @pl.when(c == 1)
        def _():
            run(g1_hbm, s1_hbm)

    call = pl.kernel(
        body,
        out_type=[jax.ShapeDtypeStruct((n, 128), _F32),
                  jax.ShapeDtypeStruct((n, 128), _F32)],
        mesh=_mesh,
        name="sc_scatter_add",
        scratch_types=[
            pltpu.VMEM_SHARED((nsp, 128), _F32),
            pltpu.VMEM((nch, _CH), jnp.int32),
            pltpu.VMEM((_CH, 128), _F32),
            pltpu.VMEM((_CH, 128), _F32),
            pltpu.SemaphoreType.DMA,
            pltpu.SemaphoreType.DMA,
        ],
    )
    return call(g0, g1, cid2, zacc)


def _sc_counts(cid2, zacc, ones, *, n, epad):
    nw = _NC * _NS
    per_w = epad // nw
    nch = per_w // _CH
    nsp = zacc.shape[0]
    rows_w = nsp // _NS
    full = n // 640
    rem = n - full * 640

    def body(cid_hbm, z_hbm, ones_hbm, ca_hbm, cb_hbm, cnt, idxv, ones):
        c = lax.axis_index("c")
        s = lax.axis_index("s")
        w = s * _NC + c
        pltpu.sync_copy(ones_hbm, ones)
        pltpu.sync_copy(z_hbm.at[pl.ds(s * rows_w, rows_w)],
                        cnt.at[pl.ds(s * rows_w, rows_w)])
        pltpu.sync_copy(cid_hbm.at[pl.ds(w * nch, nch)], idxv)
        plsc.subcore_barrier()

        def chunk(j, carry):
            pltpu.sync_copy(ones, cnt.at[idxv.at[j]], add=True)
            return carry

        lax.fori_loop(0, nch, chunk, 0)
        plsc.subcore_barrier()

        def write(out_hbm):
            @pl.when(s < full)
            def _():
                pltpu.sync_copy(cnt.at[pl.ds(s * 640, 640)],
                                out_hbm.at[pl.ds(s * 640, 640)])

            if rem:
                @pl.when(s == full)
                def _():
                    pltpu.sync_copy(cnt.at[pl.ds(full * 640, rem)],
                                    out_hbm.at[pl.ds(full * 640, rem)])

        @pl.when(c == 0)
        def _():
            write(ca_hbm)

        @pl.when(c == 1)
        def _():
            write(cb_hbm)

    call = pl.kernel(
        body,
        out_type=[jax.ShapeDtypeStruct((n, 128), _F32),
                  jax.ShapeDtypeStruct((n, 128), _F32)],
        mesh=_mesh,
        name="sc_counts",
        scratch_types=[
            pltpu.VMEM_SHARED((nsp, 128), _F32),
            pltpu.VMEM((nch, _CH), jnp.int32),
            pltpu.VMEM((_CH, 128), _F32),
        ],
    )
    return call(cid2, zacc, ones)


def _tc_edge(xr, xc, ea, Wa, Wb, Wc, eb1, ew2, eb2, Qa, Qb, n1b1, g1s, n1be):
    epad = xr.shape[0]
    ef = ea.shape[1]
    be = 2048
    grid = epad // be

    def body(xr_ref, xc_ref, ea_ref, wa_ref, wb_ref, wc_ref, eb1_ref,
             ew2_ref, eb2_ref, qa_ref, qb_ref, n1b1_ref, g1s_ref, n1be_ref,
             e_ref, g0_ref, g1_ref):
        xr_ = xr_ref[...]
        xc_ = xc_ref[...]
        dot = functools.partial(jnp.dot, preferred_element_type=_F32)
        h1 = dot(xr_, wa_ref[...]) + dot(xc_, wb_ref[...])
        h1 = h1 + dot(ea_ref[...], wc_ref[...]) + eb1_ref[...]
        h1 = jnp.maximum(h1, 0.0)
        e = dot(h1, ew2_ref[...]) + eb2_ref[...]
        z = dot(xr_, qa_ref[...]) + dot(e, qb_ref[...]) + n1b1_ref[...]
        g = jnp.maximum(z * g1s_ref[...] + n1be_ref[...], 0.0)
        e_ref[...] = e
        g0_ref[...] = g[:, :128]
        g1_ref[...] = g[:, 128:]

    full = lambda shape: pl.BlockSpec(shape, lambda i: (0, 0))
    return pl.pallas_call(
        body,
        grid=(grid,),
        in_specs=[
            pl.BlockSpec((be, 128), lambda i: (i, 0)),
            pl.BlockSpec((be, 128), lambda i: (i, 0)),
            pl.BlockSpec((be, ef), lambda i: (i, 0)),
            full(Wa.shape), full(Wb.shape), full(Wc.shape), full(eb1.shape),
            full(ew2.shape), full(eb2.shape), full(Qa.shape), full(Qb.shape),
            full(n1b1.shape), full(g1s.shape), full(n1be.shape),
        ],
        out_specs=[
            pl.BlockSpec((be, ef), lambda i: (i, 0)),
            pl.BlockSpec((be, 128), lambda i: (i, 0)),
            pl.BlockSpec((be, 128), lambda i: (i, 0)),
        ],
        out_shape=[
            jax.ShapeDtypeStruct((epad, ef), _F32),
            jax.ShapeDtypeStruct((epad, 128), _F32),
            jax.ShapeDtypeStruct((epad, 128), _F32),
        ],
    )(xr, xc, ea, Wa, Wb, Wc, eb1, ew2, eb2, Qa, Qb, n1b1, g1s, n1be)


def _tc_node(S0, S1, x, cntA, cntB, w2a, w2b, n1b2, n2w1a, n2w1b,
             n2b1, g2s, n2be, n2w2, n2b2):
    n = x.shape[0]
    bn = 1000
    grid = n // bn

    def body(s0_ref, s1_ref, x_ref, ca_ref, cb_ref, w2a_ref, w2b_ref,
             n1b2_ref, w1a_ref, w1b_ref, n2b1_ref, g2s_ref, n2be_ref,
             w22_ref, n2b2_ref, out_ref):
        cnt = ca_ref[...][:, :1] + cb_ref[...][:, :1]
        rcp = 1.0 / jnp.maximum(cnt, 1.0)
        t = (cnt > 0.0).astype(_F32)
        dot = functools.partial(jnp.dot, preferred_element_type=_F32)
        agg = dot(s0_ref[...] * rcp, w2a_ref[...])
        agg = agg + dot(s1_ref[...] * rcp, w2b_ref[...]) + t * n1b2_ref[...]
        z = dot(x_ref[...], w1a_ref[...]) + dot(agg, w1b_ref[...]) + n2b1_ref[...]
        h = jnp.maximum(z * g2s_ref[...] + n2be_ref[...], 0.0)
        out_ref[...] = dot(h, w22_ref[...]) + n2b2_ref[...]

    full = lambda shape: pl.BlockSpec(shape, lambda i: (0, 0))
    return pl.pallas_call(
        body,
        grid=(grid,),
        in_specs=[
            pl.BlockSpec((bn, 128), lambda i: (i, 0)),
            pl.BlockSpec((bn, 128), lambda i: (i, 0)),
            pl.BlockSpec((bn, 128), lambda i: (i, 0)),
            pl.BlockSpec((bn, 16), lambda i: (i, 0)),
            pl.BlockSpec((bn, 16), lambda i: (i, 0)),
            full(w2a.shape), full(w2b.shape), full(n1b2.shape),
            full(n2w1a.shape), full(n2w1b.shape), full(n2b1.shape),
            full(g2s.shape), full(n2be.shape), full(n2w2.shape),
            full(n2b2.shape),
        ],
        out_specs=[pl.BlockSpec((bn, 128), lambda i: (i, 0))],
        out_shape=[jax.ShapeDtypeStruct((n, 128), _F32)],
    )(S0, S1, x, cntA, cntB, w2a, w2b, n1b2, n2w1a, n2w1b,
      n2b1, g2s, n2be, n2w2, n2b2)[0]


def _tc_heads(x, P, pa, pn):
    def head(h0, p_refs):
        (w1, b1, g1, be1, w2, b2, g2, be2, w3, b3) = p_refs
        dot = functools.partial(jnp.dot, preferred_element_type=_F32)
        h = jnp.maximum((dot(h0, w1[...]) + b1[...]) * (_BNS * g1[...]) + be1[...], 0.0)
        h = jnp.maximum((dot(h, w2[...]) + b2[...]) * (_BNS * g2[...]) + be2[...], 0.0)
        return dot(h, w3[...]) + b3[...]

    def body(x_ref, p_ref, *refs):
        pa_refs = refs[:10]
        pn_refs = refs[10:20]
        oa_ref, on_ref = refs[20], refs[21]
        emb = jnp.dot(p_ref[...], x_ref[...], preferred_element_type=_F32)
        oa_ref[...] = head(emb, pa_refs)
        on_ref[...] = head(emb, pn_refs)

    ins = [x, P] + list(pa) + list(pn)
    return pl.pallas_call(
        body,
        out_shape=[jax.ShapeDtypeStruct((P.shape[0], pa[-2].shape[1]), _F32),
                   jax.ShapeDtypeStruct((P.shape[0], pn[-2].shape[1]), _F32)],
    )(*ins)


def _head_ops(p):
    r = lambda v: v.reshape(1, -1)
    return (p['w1'], r(p['b1']), r(p['g1']), r(p['be1']),
            p['w2'], r(p['b2']), r(p['g2']), r(p['be2']),
            p['w3'], r(p['b3']))


def kernel(x, edge_index, edge_attr, batch, params):
    n, nf = x.shape
    e = edge_index.shape[1]
    ef = edge_attr.shape[1]
    ng = 16
    step = _NC * _NS * _CH
    epad = ((e + step - 1) // step) * step
    pad = epad - e

    row = edge_index[0]
    col = edge_index[1]
    i32 = jnp.int32
    rid1 = jnp.concatenate([row, jnp.zeros((pad,), i32)])
    cidg1 = jnp.concatenate([col, jnp.zeros((pad,), i32)])
    cids2 = jnp.concatenate([col, jnp.full((pad,), n, i32)]).reshape(epad // _CH, _CH)
    ea = jnp.concatenate([edge_attr, jnp.zeros((pad, ef), _F32)], axis=0)
    nsp = ((n + 1 + 127) // 128) * 128
    zacc = jnp.zeros((nsp, 128), _F32)
    P = (batch[None, :] == jnp.arange(ng, dtype=batch.dtype)[:, None]
         ).astype(_F32) * (1.0 / (n // ng))

    ones128 = jnp.ones((_CH, 128), _F32)
    cA128, cB128 = _sc_counts(cids2, zacc, ones128, n=n, epad=epad)
    cntA, cntB = cA128[:, :16], cB128[:, :16]

    for name in ('gnn1', 'gnn2', 'gnn3'):
        p = params[name]
        r = lambda v: v.reshape(1, -1)
        xr, xc = _sc_gather(x, rid1, cidg1, epad=epad)
        e_new, g0, g1 = _tc_edge(
            xr, xc, ea,
            p['ew1'][:nf], p['ew1'][nf:2 * nf], p['ew1'][2 * nf:],
            r(p['eb1']), p['ew2'], r(p['eb2']),
            p['n1w1'][:nf], p['n1w1'][nf:], r(p['n1b1']),
            r(p['n1g']) * _BNS, r(p['n1be']))
        S0, S1 = _sc_scatter(g0, g1, cids2, zacc, n=n, epad=epad)
        x = _tc_node(
            S0, S1, x, cntA, cntB,
            p['n1w2'][:128], p['n1w2'][128:], r(p['n1b2']),
            p['n2w1'][:nf], p['n2w1'][nf:], r(p['n2b1']),
            r(p['n2g']) * _BNS, r(p['n2be']), p['n2w2'], r(p['n2b2']))
        ea = e_new

    action_prob, node_score = _tc_heads(
        x, P, _head_ops(params['action']), _head_ops(params['node']))
    return (action_prob, node_score)


# speedup vs baseline: 1.0592x; 1.0592x over previous
import functools

import jax
import jax.numpy as jnp
import numpy as np
from jax import lax
from jax.experimental import pallas as pl
from jax.experimental.pallas import tpu as pltpu
from jax.experimental.pallas import tpu_sc as plsc

_EPS = 1e-5
_BNS = float(1.0 / np.sqrt(1.0 + _EPS))
_NC = 2
_NS = 16
_CH = 128
_F32 = jnp.float32

_mesh = plsc.VectorSubcoreMesh(
    core_axis_name="c", subcore_axis_name="s", num_cores=_NC, num_subcores=_NS)


def _sc_gather(x, rid2, cid2, *, epad):
    per_sp = epad // _NS
    gch = 2 * _CH
    ntot = per_sp // gch
    ncg0 = (ntot * 31 + 39) // 40
    ncg1 = ntot - ncg0
    assert per_sp % gch == 0 and ncg1 >= 4

    def body(x_hbm, rid_hbm, cid_hbm, xr_hbm, xc_hbm,
             ridv, cidv, b0, b1, b2, s0, s1, s2):
        c = lax.axis_index("c")
        s = lax.axis_index("s")
        bufs = (b0, b1, b2)
        sems = (s0, s1, s2)

        def worker(base, ncg):
            pltpu.sync_copy(rid_hbm.at[pl.ds(base, ncg * gch)],
                            ridv.at[pl.ds(0, ncg * gch)])
            pltpu.sync_copy(cid_hbm.at[pl.ds(base, ncg * gch)],
                            cidv.at[pl.ds(0, ncg * gch)])

            def phase(idxref, out_hbm):
                def issue(t, b):
                    pltpu.async_copy(x_hbm.at[idxref.at[pl.ds(gch * t, gch)]],
                                     bufs[b], sems[b])

                def drain(t, b):
                    pltpu.make_async_copy(
                        x_hbm.at[idxref.at[pl.ds(gch * t, gch)]],
                        bufs[b], sems[b]).wait()
                    pltpu.sync_copy(bufs[b],
                                    out_hbm.at[pl.ds(base + gch * t, gch)])

                for b in range(3):
                    issue(b, b)
                main = (ncg - 3) // 3

                def ring(tt, carry):
                    for b in range(3):
                        t = 3 * tt + b
                        drain(t, b)
                        issue(t + 3, b)
                    return carry

                lax.fori_loop(0, main, ring, 0)
                nxt = 3 * main + 3
                for t in range(3 * main, ncg):
                    b = t % 3
                    drain(t, b)
                    if nxt < ncg:
                        issue(nxt, b)
                        nxt += 1

            phase(ridv, xr_hbm)
            phase(cidv, xc_hbm)

        @pl.when(c == 0)
        def _():
            worker(s * per_sp, ncg0)

        @pl.when(c == 1)
        def _():
            worker(s * per_sp + ncg0 * gch, ncg1)

    call = pl.kernel(
        body,
        out_type=[jax.ShapeDtypeStruct((epad, 128), _F32),
                  jax.ShapeDtypeStruct((epad, 128), _F32)],
        mesh=_mesh,
        name="sc_gather",
        scratch_types=[
            pltpu.VMEM((ncg0 * gch,), jnp.int32),
            pltpu.VMEM((ncg0 * gch,), jnp.int32),
            pltpu.VMEM((gch, 128), _F32),
            pltpu.VMEM((gch, 128), _F32),
            pltpu.VMEM((gch, 128), _F32),
            pltpu.SemaphoreType.DMA,
            pltpu.SemaphoreType.DMA,
            pltpu.SemaphoreType.DMA,
        ],
    )
    return call(x, rid2, cid2)


def _sc_scatter(g0, g1, cid2, zacc, *, n, epad):
    per_s = epad // _NS
    nch = per_s // _CH
    nsp = zacc.shape[0]
    rows_w = nsp // _NS
    full = n // 640
    rem = n - full * 640

    def body(g0_hbm, g1_hbm, cid_hbm, z_hbm, s0_hbm, s1_hbm,
             acc, idxv, bufa, bufb, sema, semb):
        c = lax.axis_index("c")
        s = lax.axis_index("s")
        pltpu.sync_copy(z_hbm.at[pl.ds(s * rows_w, rows_w)],
                        acc.at[pl.ds(s * rows_w, rows_w)])
        pltpu.sync_copy(cid_hbm.at[pl.ds(s * nch, nch)], idxv)
        plsc.subcore_barrier()

        def run(g_hbm, out_hbm):
            base = s * per_s

            def pair(jj, carry):
                r0 = base + (2 * jj) * _CH
                ca = pltpu.async_copy(g_hbm.at[pl.ds(r0, _CH)], bufa, sema)
                cb = pltpu.async_copy(g_hbm.at[pl.ds(r0 + _CH, _CH)], bufb, semb)
                ca.wait()
                pltpu.sync_copy(bufa, acc.at[idxv.at[2 * jj]], add=True)
                cb.wait()
                pltpu.sync_copy(bufb, acc.at[idxv.at[2 * jj + 1]], add=True)
                return carry

            lax.fori_loop(0, nch // 2, pair, 0)
            plsc.subcore_barrier()

            @pl.when(s < full)
            def _():
                pltpu.sync_copy(acc.at[pl.ds(s * 640, 640)],
                                out_hbm.at[pl.ds(s * 640, 640)])

            if rem:
                @pl.when(s == full)
                def _():
                    pltpu.sync_copy(acc.at[pl.ds(full * 640, rem)],
                                    out_hbm.at[pl.ds(full * 640, rem)])

        @pl.when(c == 0)
        def _():
            run(g0_hbm, s0_hbm)

        @pl.when(c == 1)
        def _():
            run(g1_hbm, s1_hbm)

    call = pl.kernel(
        body,
        out_type=[jax.ShapeDtypeStruct((n, 128), _F32),
                  jax.ShapeDtypeStruct((n, 128), _F32)],
        mesh=_mesh,
        name="sc_scatter_add",
        scratch_types=[
            pltpu.VMEM_SHARED((nsp, 128), _F32),
            pltpu.VMEM((nch, _CH), jnp.int32),
            pltpu.VMEM((_CH, 128), _F32),
            pltpu.VMEM((_CH, 128), _F32),
            pltpu.SemaphoreType.DMA,
            pltpu.SemaphoreType.DMA,
        ],
    )
    return call(g0, g1, cid2, zacc)


def _sc_counts(cid2, zacc, ones, *, n, epad):
    nw = _NC * _NS
    per_w = epad // nw
    nch = per_w // _CH
    nsp = zacc.shape[0]
    rows_w = nsp // _NS
    full = n // 640
    rem = n - full * 640

    def body(cid_hbm, z_hbm, ones_hbm, ca_hbm, cb_hbm, cnt, idxv, ones):
        c = lax.axis_index("c")
        s = lax.axis_index("s")
        w = s * _NC + c
        pltpu.sync_copy(ones_hbm, ones)
        pltpu.sync_copy(z_hbm.at[pl.ds(s * rows_w, rows_w)],
                        cnt.at[pl.ds(s * rows_w, rows_w)])
        pltpu.sync_copy(cid_hbm.at[pl.ds(w * nch, nch)], idxv)
        plsc.subcore_barrier()

        def chunk(j, carry):
            pltpu.sync_copy(ones, cnt.at[idxv.at[j]], add=True)
            return carry

        lax.fori_loop(0, nch, chunk, 0)
        plsc.subcore_barrier()

        def write(out_hbm):
            @pl.when(s < full)
            def _():
                pltpu.sync_copy(cnt.at[pl.ds(s * 640, 640)],
                                out_hbm.at[pl.ds(s * 640, 640)])

            if rem:
                @pl.when(s == full)
                def _():
                    pltpu.sync_copy(cnt.at[pl.ds(full * 640, rem)],
                                    out_hbm.at[pl.ds(full * 640, rem)])

        @pl.when(c == 0)
        def _():
            write(ca_hbm)

        @pl.when(c == 1)
        def _():
            write(cb_hbm)

    call = pl.kernel(
        body,
        out_type=[jax.ShapeDtypeStruct((n, 128), _F32),
                  jax.ShapeDtypeStruct((n, 128), _F32)],
        mesh=_mesh,
        name="sc_counts",
        scratch_types=[
            pltpu.VMEM_SHARED((nsp, 128), _F32),
            pltpu.VMEM((nch, _CH), jnp.int32),
            pltpu.VMEM((_CH, 128), _F32),
        ],
    )
    return call(cid2, zacc, ones)


def _tc_edge(xr, xc, ea, Wa, Wb, Wc, eb1, ew2, eb2, Qa, Qb, n1b1, g1s, n1be):
    epad = xr.shape[0]
    ef = ea.shape[1]
    be = 2048
    grid = epad // be

    def body(xr_ref, xc_ref, ea_ref, wa_ref, wb_ref, wc_ref, eb1_ref,
             ew2_ref, eb2_ref, qa_ref, qb_ref, n1b1_ref, g1s_ref, n1be_ref,
             e_ref, g0_ref, g1_ref):
        xr_ = xr_ref[...]
        xc_ = xc_ref[...]
        dot = functools.partial(jnp.dot, preferred_element_type=_F32)
        h1 = dot(xr_, wa_ref[...]) + dot(xc_, wb_ref[...])
        h1 = h1 + dot(ea_ref[...], wc_ref[...]) + eb1_ref[...]
        h1 = jnp.maximum(h1, 0.0)
        e = dot(h1, ew2_ref[...]) + eb2_ref[...]
        z = dot(xr_, qa_ref[...]) + dot(e, qb_ref[...]) + n1b1_ref[...]
        g = jnp.maximum(z * g1s_ref[...] + n1be_ref[...], 0.0)
        e_ref[...] = e
        g0_ref[...] = g[:, :128]
        g1_ref[...] = g[:, 128:]

    full = lambda shape: pl.BlockSpec(shape, lambda i: (0, 0))
    return pl.pallas_call(
        body,
        grid=(grid,),
        in_specs=[
            pl.BlockSpec((be, 128), lambda i: (i, 0)),
            pl.BlockSpec((be, 128), lambda i: (i, 0)),
            pl.BlockSpec((be, ef), lambda i: (i, 0)),
            full(Wa.shape), full(Wb.shape), full(Wc.shape), full(eb1.shape),
            full(ew2.shape), full(eb2.shape), full(Qa.shape), full(Qb.shape),
            full(n1b1.shape), full(g1s.shape), full(n1be.shape),
        ],
        out_specs=[
            pl.BlockSpec((be, ef), lambda i: (i, 0)),
            pl.BlockSpec((be, 128), lambda i: (i, 0)),
            pl.BlockSpec((be, 128), lambda i: (i, 0)),
        ],
        out_shape=[
            jax.ShapeDtypeStruct((epad, ef), _F32),
            jax.ShapeDtypeStruct((epad, 128), _F32),
            jax.ShapeDtypeStruct((epad, 128), _F32),
        ],
    )(xr, xc, ea, Wa, Wb, Wc, eb1, ew2, eb2, Qa, Qb, n1b1, g1s, n1be)


def _tc_node(S0, S1, x, cntA, cntB, w2a, w2b, n1b2, n2w1a, n2w1b,
             n2b1, g2s, n2be, n2w2, n2b2):
    n = x.shape[0]
    bn = 1000
    grid = n // bn

    def body(s0_ref, s1_ref, x_ref, ca_ref, cb_ref, w2a_ref, w2b_ref,
             n1b2_ref, w1a_ref, w1b_ref, n2b1_ref, g2s_ref, n2be_ref,
             w22_ref, n2b2_ref, out_ref):
        cnt = ca_ref[...][:, :1] + cb_ref[...][:, :1]
        rcp = 1.0 / jnp.maximum(cnt, 1.0)
        t = (cnt > 0.0).astype(_F32)
        dot = functools.partial(jnp.dot, preferred_element_type=_F32)
        agg = dot(s0_ref[...] * rcp, w2a_ref[...])
        agg = agg + dot(s1_ref[...] * rcp, w2b_ref[...]) + t * n1b2_ref[...]
        z = dot(x_ref[...], w1a_ref[...]) + dot(agg, w1b_ref[...]) + n2b1_ref[...]
        h = jnp.maximum(z * g2s_ref[...] + n2be_ref[...], 0.0)
        out_ref[...] = dot(h, w22_ref[...]) + n2b2_ref[...]

    full = lambda shape: pl.BlockSpec(shape, lambda i: (0, 0))
    return pl.pallas_call(
        body,
        grid=(grid,),
        in_specs=[
            pl.BlockSpec((bn, 128), lambda i: (i, 0)),
            pl.BlockSpec((bn, 128), lambda i: (i, 0)),
            pl.BlockSpec((bn, 128), lambda i: (i, 0)),
            pl.BlockSpec((bn, 16), lambda i: (i, 0)),
            pl.BlockSpec((bn, 16), lambda i: (i, 0)),
            full(w2a.shape), full(w2b.shape), full(n1b2.shape),
            full(n2w1a.shape), full(n2w1b.shape), full(n2b1.shape),
            full(g2s.shape), full(n2be.shape), full(n2w2.shape),
            full(n2b2.shape),
        ],
        out_specs=[pl.BlockSpec((bn, 128), lambda i: (i, 0))],
        out_shape=[jax.ShapeDtypeStruct((n, 128), _F32)],
    )(S0, S1, x, cntA, cntB, w2a, w2b, n1b2, n2w1a, n2w1b,
      n2b1, g2s, n2be, n2w2, n2b2)[0]


def _tc_heads(x, P, pa, pn):
    def head(h0, p_refs):
        (w1, b1, g1, be1, w2, b2, g2, be2, w3, b3) = p_refs
        dot = functools.partial(jnp.dot, preferred_element_type=_F32)
        h = jnp.maximum((dot(h0, w1[...]) + b1[...]) * (_BNS * g1[...]) + be1[...], 0.0)
        h = jnp.maximum((dot(h, w2[...]) + b2[...]) * (_BNS * g2[...]) + be2[...], 0.0)
        return dot(h, w3[...]) + b3[...]

    def body(x_ref, p_ref, *refs):
        pa_refs = refs[:10]
        pn_refs = refs[10:20]
        oa_ref, on_ref = refs[20], refs[21]
        emb = jnp.dot(p_ref[...], x_ref[...], preferred_element_type=_F32)
        oa_ref[...] = head(emb, pa_refs)
        on_ref[...] = head(emb, pn_refs)

    ins = [x, P] + list(pa) + list(pn)
    return pl.pallas_call(
        body,
        out_shape=[jax.ShapeDtypeStruct((P.shape[0], pa[-2].shape[1]), _F32),
                   jax.ShapeDtypeStruct((P.shape[0], pn[-2].shape[1]), _F32)],
    )(*ins)


def _head_ops(p):
    r = lambda v: v.reshape(1, -1)
    return (p['w1'], r(p['b1']), r(p['g1']), r(p['be1']),
            p['w2'], r(p['b2']), r(p['g2']), r(p['be2']),
            p['w3'], r(p['b3']))


def kernel(x, edge_index, edge_attr, batch, params):
    n, nf = x.shape
    e = edge_index.shape[1]
    ef = edge_attr.shape[1]
    ng = 16
    step = _NC * _NS * _CH
    epad = ((e + step - 1) // step) * step
    pad = epad - e

    row = edge_index[0]
    col = edge_index[1]
    i32 = jnp.int32
    rid1 = jnp.concatenate([row, jnp.zeros((pad,), i32)])
    cidg1 = jnp.concatenate([col, jnp.zeros((pad,), i32)])
    cids2 = jnp.concatenate([col, jnp.full((pad,), n, i32)]).reshape(epad // _CH, _CH)
    ea = jnp.concatenate([edge_attr, jnp.zeros((pad, ef), _F32)], axis=0)
    nsp = ((n + 1 + 127) // 128) * 128
    zacc = jnp.zeros((nsp, 128), _F32)
    P = (batch[None, :] == jnp.arange(ng, dtype=batch.dtype)[:, None]
         ).astype(_F32) * (1.0 / (n // ng))

    ones128 = jnp.ones((_CH, 128), _F32)
    cA128, cB128 = _sc_counts(cids2, zacc, ones128, n=n, epad=epad)
    cntA, cntB = cA128[:, :16], cB128[:, :16]

    for name in ('gnn1', 'gnn2', 'gnn3'):
        p = params[name]
        r = lambda v: v.reshape(1, -1)
        xr, xc = _sc_gather(x, rid1, cidg1, epad=epad)
        e_new, g0, g1 = _tc_edge(
            xr, xc, ea,
            p['ew1'][:nf], p['ew1'][nf:2 * nf], p['ew1'][2 * nf:],
            r(p['eb1']), p['ew2'], r(p['eb2']),
            p['n1w1'][:nf], p['n1w1'][nf:], r(p['n1b1']),
            r(p['n1g']) * _BNS, r(p['n1be']))
        S0, S1 = _sc_scatter(g0, g1, cids2, zacc, n=n, epad=epad)
        x = _tc_node(
            S0, S1, x, cntA, cntB,
            p['n1w2'][:128], p['n1w2'][128:], r(p['n1b2']),
            p['n2w1'][:nf], p['n2w1'][nf:], r(p['n2b1']),
            r(p['n2g']) * _BNS, r(p['n2be']), p['n2w2'], r(p['n2b2']))
        ea = e_new

    action_prob, node_score = _tc_heads(
        x, P, _head_ops(params['action']), _head_ops(params['node']))
    return (action_prob, node_score)
